# baseline (speedup 1.0000x reference)
import jax
import jax.numpy as jnp
import numpy as np
from jax import lax
from jax.experimental import pallas as pl
from jax.experimental.pallas import tpu as pltpu
from jax.experimental.pallas import tpu_sc as plsc

B, R, G, D = 4, 8, 10000, 128
DP = D
NC, NS, L = 2, 16, 16
NW = NC * NS
CG = 50
CHUNKS_PER_B = G // CG
CHUNKS = B * CHUNKS_PER_B
BASE_CHUNKS = CHUNKS // NW
EXTRA = CHUNKS % NW
PAIRS = (BASE_CHUNKS + 2) // 2
UNROLL = 2


def _chunk_coords(c):
    b = c // CHUNKS_PER_B
    g0 = (c % CHUNKS_PER_B) * CG
    return b, g0


def _issue_in(x_hbm, c, buf, sem):
    b, g0 = _chunk_coords(c)
    for r in range(R):
        pltpu.async_copy(
            x_hbm.at[b, r, pl.ds(g0, CG)], buf.at[r, :, pl.ds(0, D)], sem
        )


def _wait_in(x_hbm, c, buf, sem):
    b, g0 = _chunk_coords(c)
    for r in range(R):
        pltpu.make_async_copy(
            x_hbm.at[b, r, pl.ds(g0, CG)], buf.at[r, :, pl.ds(0, D)], sem
        ).wait()


def _out_copy(out_hbm, c, outbuf, sem):
    b, g0 = _chunk_coords(c)
    return pltpu.make_async_copy(
        outbuf.at[:, pl.ds(0, D)], out_hbm.at[b, pl.ds(g0, CG)], sem
    )


def _compute_chunk(buf, outbuf):
    lane_xor = [lax.iota(jnp.int32, L) ^ off for off in (8, 4, 2, 1)]

    @plsc.parallel_loop(0, CG, 1, unroll=UNROLL)
    def g_body(g):
        best = jnp.full((L,), -1.0, jnp.float32)
        best_r = jnp.zeros((L,), jnp.int32)
        for r in range(R):
            q = []
            for k in range(D // L):
                v = buf[r, g, pl.ds(k * L, L)]
                q.append(v * v)
            a = [q[k] + q[k + 4] for k in range(4)]
            b = [a[k] + a[k + 2] for k in range(2)]
            c = b[0] + b[1]
            for idx in lane_xor:
                c = c + c.at[idx].get(mode="promise_in_bounds")
            upd = c > best
            best = jnp.where(upd, c, best)
            best_r = jnp.where(upd, jnp.full((L,), r, jnp.int32), best_r)
        w = best_r[0]
        for k in range(D // L):
            outbuf[g, pl.ds(k * L, L)] = buf[w, g, pl.ds(k * L, L)]


def _body(x_hbm, out_hbm, buf0, buf1, outbuf0, outbuf1,
          sem_i0, sem_i1, sem_o0, sem_o1):
    wid = lax.axis_index("s") * NC + lax.axis_index("c")
    n = jnp.where(wid < EXTRA, BASE_CHUNKS + 1, BASE_CHUNKS)

    _issue_in(x_hbm, wid, buf0, sem_i0)

    def pair_body(k, carry):
        c0 = wid + (2 * k) * NW
        c1 = wid + (2 * k + 1) * NW

        _wait_in(x_hbm, c0, buf0, sem_i0)

        @pl.when(2 * k + 1 < n)
        def _():
            _issue_in(x_hbm, c1, buf1, sem_i1)

        @pl.when(k >= 1)
        def _():
            _out_copy(out_hbm, c0 - 2 * NW, outbuf0, sem_o0).wait()

        _compute_chunk(buf0, outbuf0)
        _out_copy(out_hbm, c0, outbuf0, sem_o0).start()

        @pl.when(2 * k + 1 < n)
        def _():
            _wait_in(x_hbm, c1, buf1, sem_i1)

            @pl.when(2 * k + 2 < n)
            def _():
                _issue_in(x_hbm, c1 + NW, buf0, sem_i0)

            @pl.when(k >= 1)
            def _():
                _out_copy(out_hbm, c1 - 2 * NW, outbuf1, sem_o1).wait()

            _compute_chunk(buf1, outbuf1)
            _out_copy(out_hbm, c1, outbuf1, sem_o1).start()

        return carry

    lax.fori_loop(0, PAIRS, pair_body, 0)

    _out_copy(out_hbm, wid, outbuf0, sem_o0).wait()
    _out_copy(out_hbm, wid, outbuf1, sem_o1).wait()


@jax.jit
def kernel(inputs):
    mesh = plsc.VectorSubcoreMesh(core_axis_name="c", subcore_axis_name="s")
    f = pl.kernel(
        _body,
        out_type=jax.ShapeDtypeStruct((B, G, D), jnp.float32),
        mesh=mesh,
        scratch_types=[
            pltpu.VMEM((R, CG, DP), jnp.float32),
            pltpu.VMEM((R, CG, DP), jnp.float32),
            pltpu.VMEM((CG, DP), jnp.float32),
            pltpu.VMEM((CG, DP), jnp.float32),
            pltpu.SemaphoreType.DMA,
            pltpu.SemaphoreType.DMA,
            pltpu.SemaphoreType.DMA,
            pltpu.SemaphoreType.DMA,
        ],
        compiler_params=pltpu.CompilerParams(
            use_tc_tiling_on_sc=False, needs_layout_passes=False
        ),
    )
    return f(inputs)


# speedup vs baseline: 1.1132x; 1.1132x over previous
import jax
import jax.numpy as jnp
import numpy as np
from jax import lax
from jax.experimental import pallas as pl
from jax.experimental.pallas import tpu as pltpu
from jax.experimental.pallas import tpu_sc as plsc

B, R, G, D = 4, 8, 10000, 128
DP = D
NC, NS, L = 2, 16, 16
NW = NC * NS
CG = 50
CHUNKS_PER_B = G // CG
CHUNKS = B * CHUNKS_PER_B
BASE_CHUNKS = CHUNKS // NW
EXTRA = CHUNKS % NW
PAIRS = (BASE_CHUNKS + 2) // 2
UNROLL = 1


def _chunk_coords(c):
    b = c // CHUNKS_PER_B
    g0 = (c % CHUNKS_PER_B) * CG
    return b, g0


def _issue_in(x_hbm, c, buf, sem):
    b, g0 = _chunk_coords(c)
    for r in range(R):
        pltpu.async_copy(
            x_hbm.at[b, r, pl.ds(g0, CG)], buf.at[r, :, pl.ds(0, D)], sem
        )


def _wait_in(x_hbm, c, buf, sem):
    b, g0 = _chunk_coords(c)
    for r in range(R):
        pltpu.make_async_copy(
            x_hbm.at[b, r, pl.ds(g0, CG)], buf.at[r, :, pl.ds(0, D)], sem
        ).wait()


def _out_copy(out_hbm, c, outbuf, sem):
    b, g0 = _chunk_coords(c)
    return pltpu.make_async_copy(
        outbuf.at[:, pl.ds(0, D)], out_hbm.at[b, pl.ds(g0, CG)], sem
    )


def _compute_chunk(buf, outbuf):
    lane_xor = [lax.iota(jnp.int32, L) ^ off for off in (8, 4, 2, 1)]

    @plsc.parallel_loop(0, CG, 1, unroll=UNROLL)
    def g_body(g):
        best = jnp.full((L,), -1.0, jnp.float32)
        best_r = jnp.zeros((L,), jnp.int32)
        for r in range(R):
            q = []
            for k in range(D // L):
                v = buf[r, g, pl.ds(k * L, L)]
                q.append(v * v)
            a = [q[k] + q[k + 4] for k in range(4)]
            b = [a[k] + a[k + 2] for k in range(2)]
            c = b[0] + b[1]
            for idx in lane_xor:
                c = c + c.at[idx].get(mode="promise_in_bounds")
            upd = c > best
            best = jnp.where(upd, c, best)
            best_r = jnp.where(upd, jnp.full((L,), r, jnp.int32), best_r)
        w = best_r[0]
        for k in range(D // L):
            outbuf[g, pl.ds(k * L, L)] = buf[w, g, pl.ds(k * L, L)]


def _body(x_hbm, out_hbm, buf0, buf1, outbuf0, outbuf1,
          sem_i0, sem_i1, sem_o0, sem_o1):
    wid = lax.axis_index("s") * NC + lax.axis_index("c")
    n = jnp.where(wid < EXTRA, BASE_CHUNKS + 1, BASE_CHUNKS)

    _issue_in(x_hbm, wid, buf0, sem_i0)

    def pair_body(k, carry):
        c0 = wid + (2 * k) * NW
        c1 = wid + (2 * k + 1) * NW

        _wait_in(x_hbm, c0, buf0, sem_i0)

        @pl.when(2 * k + 1 < n)
        def _():
            _issue_in(x_hbm, c1, buf1, sem_i1)

        @pl.when(k >= 1)
        def _():
            _out_copy(out_hbm, c0 - 2 * NW, outbuf0, sem_o0).wait()

        _compute_chunk(buf0, outbuf0)
        _out_copy(out_hbm, c0, outbuf0, sem_o0).start()

        @pl.when(2 * k + 1 < n)
        def _():
            _wait_in(x_hbm, c1, buf1, sem_i1)

            @pl.when(2 * k + 2 < n)
            def _():
                _issue_in(x_hbm, c1 + NW, buf0, sem_i0)

            @pl.when(k >= 1)
            def _():
                _out_copy(out_hbm, c1 - 2 * NW, outbuf1, sem_o1).wait()

            _compute_chunk(buf1, outbuf1)
            _out_copy(out_hbm, c1, outbuf1, sem_o1).start()

        return carry

    lax.fori_loop(0, PAIRS, pair_body, 0)

    _out_copy(out_hbm, wid, outbuf0, sem_o0).wait()
    _out_copy(out_hbm, wid, outbuf1, sem_o1).wait()


@jax.jit
def kernel(inputs):
    mesh = plsc.VectorSubcoreMesh(core_axis_name="c", subcore_axis_name="s")
    f = pl.kernel(
        _body,
        out_type=jax.ShapeDtypeStruct((B, G, D), jnp.float32),
        mesh=mesh,
        scratch_types=[
            pltpu.VMEM((R, CG, DP), jnp.float32),
            pltpu.VMEM((R, CG, DP), jnp.float32),
            pltpu.VMEM((CG, DP), jnp.float32),
            pltpu.VMEM((CG, DP), jnp.float32),
            pltpu.SemaphoreType.DMA,
            pltpu.SemaphoreType.DMA,
            pltpu.SemaphoreType.DMA,
            pltpu.SemaphoreType.DMA,
        ],
        compiler_params=pltpu.CompilerParams(
            use_tc_tiling_on_sc=False, needs_layout_passes=False
        ),
    )
    return f(inputs)
